# Initial kernel scaffold; baseline (speedup 1.0000x reference)
import jax
import jax.numpy as jnp
from jax.experimental import pallas as pl


def kernel(user_id, item_id, content_embedding, user_table, item_table, Wu1, bu1, Wu2, bu2, Wi1, bi1, Wi2, bi2, temperature):
    raise NotImplementedError("write your pallas kernel here")


# speedup vs baseline: 2.6086x; 2.6086x over previous
import functools

import jax
import jax.numpy as jnp
from jax import lax
from jax.experimental import pallas as pl
from jax.experimental.pallas import tpu as pltpu
from jax.experimental.pallas import tpu_sc as plsc

B = 16384
D = 128
CONTENT = 384

_NC = 2
_NS = 16
_NW = _NC * _NS
_B_PER_W = B // _NW


def _sc_gather(user_table, item_table, user_id, item_id):
    mesh = plsc.VectorSubcoreMesh(core_axis_name="c", subcore_axis_name="s")

    @functools.partial(
        pl.kernel,
        mesh=mesh,
        out_type=(
            jax.ShapeDtypeStruct((B, D), jnp.float32),
            jax.ShapeDtypeStruct((B, D), jnp.float32),
        ),
        scratch_types=[
            pltpu.VMEM((_B_PER_W,), jnp.int32),
            pltpu.VMEM((_B_PER_W, D), jnp.float32),
            pltpu.SemaphoreType.DMA,
        ],
    )
    def k(ut_hbm, it_hbm, uid_hbm, iid_hbm, uo_hbm, io_hbm, idx_v, rows_v, sem):
        wid = lax.axis_index("s") * _NC + lax.axis_index("c")
        base = wid * _B_PER_W
        pltpu.sync_copy(uid_hbm.at[pl.ds(base, _B_PER_W)], idx_v)
        pltpu.async_copy(ut_hbm.at[idx_v], rows_v, sem).wait()
        pltpu.sync_copy(rows_v, uo_hbm.at[pl.ds(base, _B_PER_W)])
        pltpu.sync_copy(iid_hbm.at[pl.ds(base, _B_PER_W)], idx_v)
        pltpu.async_copy(it_hbm.at[idx_v], rows_v, sem).wait()
        pltpu.sync_copy(rows_v, io_hbm.at[pl.ds(base, _B_PER_W)])

    return k(user_table, item_table, user_id, item_id)


def _towers_body(u_ref, it_ref, c_ref, wu1_ref, bu1_ref, wu2_ref, bu2_ref,
                 wi1a_ref, wi1b_ref, bi1_ref, wi2_ref, bi2_ref, t_ref, o_ref):
    f32 = jnp.float32
    hu = jnp.dot(u_ref[...], wu1_ref[...], preferred_element_type=f32)
    hu = jnp.maximum(hu + bu1_ref[...], 0.0)
    uv = jnp.dot(hu, wu2_ref[...], preferred_element_type=f32) + bu2_ref[...]
    uv = uv * lax.rsqrt(jnp.maximum(jnp.sum(uv * uv, axis=1, keepdims=True), 1e-12))
    hi = jnp.dot(it_ref[...], wi1a_ref[...], preferred_element_type=f32)
    hi = hi + jnp.dot(c_ref[...], wi1b_ref[...], preferred_element_type=f32)
    hi = jnp.maximum(hi + bi1_ref[...], 0.0)
    iv2 = jnp.dot(hi, wi2_ref[...], preferred_element_type=f32) + bi2_ref[...]
    iv2 = iv2 * lax.rsqrt(jnp.maximum(jnp.sum(iv2 * iv2, axis=1, keepdims=True), 1e-12))
    sim = jnp.sum(uv * iv2, axis=1, keepdims=True)
    o_ref[...] = jax.nn.sigmoid(sim / t_ref[0, 0])


def _towers(u_rows, i_rows, content, Wu1, bu1, Wu2, bu2, Wi1a, Wi1b, bi1,
            Wi2, bi2, temperature, bm=2048, interpret=False):
    grid = (B // bm,)
    row = lambda i: (i, 0)
    full = lambda i: (0, 0)
    return pl.pallas_call(
        _towers_body,
        grid=grid,
        in_specs=[
            pl.BlockSpec((bm, D), row),
            pl.BlockSpec((bm, D), row),
            pl.BlockSpec((bm, CONTENT), row),
            pl.BlockSpec((D, 128), full),
            pl.BlockSpec((1, 128), full),
            pl.BlockSpec((128, D), full),
            pl.BlockSpec((1, D), full),
            pl.BlockSpec((D, 256), full),
            pl.BlockSpec((CONTENT, 256), full),
            pl.BlockSpec((1, 256), full),
            pl.BlockSpec((256, D), full),
            pl.BlockSpec((1, D), full),
            pl.BlockSpec((1, 1), full),
        ],
        out_specs=pl.BlockSpec((bm, 1), row),
        out_shape=jax.ShapeDtypeStruct((B, 1), jnp.float32),
        interpret=interpret,
    )(u_rows, i_rows, content, Wu1, bu1, Wu2, bu2, Wi1a, Wi1b, bi1,
      Wi2, bi2, temperature)


@jax.jit
def kernel(user_id, item_id, content_embedding, user_table, item_table,
           Wu1, bu1, Wu2, bu2, Wi1, bi1, Wi2, bi2, temperature):
    uid = jnp.asarray(user_id, jnp.int32)
    iid = jnp.asarray(item_id, jnp.int32)
    u_rows, i_rows = _sc_gather(user_table, item_table, uid, iid)
    return _towers(
        u_rows, i_rows, content_embedding,
        Wu1, bu1.reshape(1, -1), Wu2, bu2.reshape(1, -1),
        Wi1[:D], Wi1[D:], bi1.reshape(1, -1), Wi2, bi2.reshape(1, -1),
        temperature.reshape(1, 1),
    )
